# Initial kernel scaffold; baseline (speedup 1.0000x reference)
import jax
import jax.numpy as jnp
from jax.experimental import pallas as pl


def kernel(in_degrees, out_degrees, in_table, out_table):
    raise NotImplementedError("write your pallas kernel here")


# speedup vs baseline: 1.0849x; 1.0849x over previous
import functools

import jax
import jax.numpy as jnp
from jax import lax
from jax.experimental import pallas as pl
from jax.experimental.pallas import tpu as pltpu, tpu_sc as plsc

MAX_DEG = 512
D = 128
N_NODES = 100000

NC = 2
NS = 16
NW = NC * NS

CHUNK = 128
N_PAD = 102400
CHUNKS_PER_W = N_PAD // (NW * CHUNK)
B_PER_W = N_PAD // NW


def _body(in_deg, out_deg, in_table, out_table, out_hbm,
          idx_a, idx_b, rows_a, rows_b, sem_a, sem_b):
    c = lax.axis_index("c")
    s = lax.axis_index("s")
    wid = s * NC + c
    base_w = wid * B_PER_W

    def chunk_body(k, carry):
        base = base_w + k * CHUNK
        pltpu.sync_copy(in_deg.at[pl.ds(base, CHUNK)], idx_a)
        pltpu.sync_copy(out_deg.at[pl.ds(base, CHUNK)], idx_b)
        for j in range(CHUNK // 16):
            sl = pl.ds(j * 16, 16)
            idx_a[sl] = jnp.clip(idx_a[sl], 0, MAX_DEG)
            idx_b[sl] = jnp.clip(idx_b[sl], 0, MAX_DEG)
        cp_a = pltpu.async_copy(in_table.at[idx_a], rows_a, sem_a)
        cp_b = pltpu.async_copy(out_table.at[idx_b], rows_b, sem_b)
        cp_a.wait()
        cp_b.wait()

        def add_row(i, cc):
            for j in range(D // 16):
                sl = pl.ds(j * 16, 16)
                rows_a[i, sl] = rows_a[i, sl] + rows_b[i, sl]
            return cc

        lax.fori_loop(0, CHUNK, add_row, 0, unroll=2)
        pltpu.sync_copy(rows_a, out_hbm.at[pl.ds(base, CHUNK)])
        return carry

    lax.fori_loop(0, CHUNKS_PER_W, chunk_body, 0)


@jax.jit
def _encode(in_deg, out_deg, in_table, out_table):
    mesh = plsc.VectorSubcoreMesh(core_axis_name="c", subcore_axis_name="s")
    kfn = pl.kernel(
        _body,
        out_type=jax.ShapeDtypeStruct((N_PAD, D), jnp.float32),
        mesh=mesh,
        scratch_types=[
            pltpu.VMEM((CHUNK,), jnp.int32),
            pltpu.VMEM((CHUNK,), jnp.int32),
            pltpu.VMEM((CHUNK, D), jnp.float32),
            pltpu.VMEM((CHUNK, D), jnp.float32),
            pltpu.SemaphoreType.DMA,
            pltpu.SemaphoreType.DMA,
        ],
    )
    return kfn(in_deg, out_deg, in_table, out_table)


def kernel(in_degrees, out_degrees, in_table, out_table):
    pad = N_PAD - N_NODES
    in_deg = jnp.pad(in_degrees.astype(jnp.int32), (0, pad))
    out_deg = jnp.pad(out_degrees.astype(jnp.int32), (0, pad))
    out = _encode(in_deg, out_deg, in_table, out_table)
    return out[:N_NODES]


# speedup vs baseline: 2.2379x; 2.0628x over previous
import jax
import jax.numpy as jnp
from jax import lax
from jax.experimental import pallas as pl
from jax.experimental.pallas import tpu as pltpu, tpu_sc as plsc

MAX_DEG = 512
D = 128
N_NODES = 100000

NC = 2
NS = 16
NW = NC * NS
L = 16

CHUNK = 128
K = 25
LAST_BASE = N_NODES - CHUNK


def _body(in_deg, out_deg, in_table, out_table, out_hbm, *scr):
    idx = [(scr[0], scr[1]), (scr[2], scr[3])]
    rows = [(scr[4], scr[5]), (scr[6], scr[7])]
    sem_g = [scr[8], scr[9]]
    sem_w = [scr[10], scr[11]]

    wid = lax.axis_index("s") * NC + lax.axis_index("c")

    pending = [None, None]
    wpending = [None, None]

    def base_of(k):
        return lax.min((k * NW + wid) * CHUNK, LAST_BASE)

    def stage(k):
        b = k % 2
        ia, ib = idx[b]
        ra, rb = rows[b]
        if wpending[b] is not None:
            wpending[b].wait()
            wpending[b] = None
        base = base_of(k)
        pltpu.sync_copy(in_deg.at[pl.ds(base, CHUNK)], ia)
        pltpu.sync_copy(out_deg.at[pl.ds(base, CHUNK)], ib)
        for j in range(CHUNK // L):
            sl = pl.ds(j * L, L)
            ia[sl] = jnp.clip(ia[sl], 0, MAX_DEG)
            ib[sl] = jnp.clip(ib[sl], 0, MAX_DEG)
        cpa = pltpu.async_copy(in_table.at[ia], ra, sem_g[b])
        cpb = pltpu.async_copy(out_table.at[ib], rb, sem_g[b])
        pending[b] = (cpa, cpb)

    stage(0)
    for k in range(K):
        b = k % 2
        ra, rb = rows[b]
        if k + 1 < K:
            stage(k + 1)
        cpa, cpb = pending[b]
        cpa.wait()
        cpb.wait()

        def add_row(i, carry):
            for j in range(D // L):
                sl = pl.ds(j * L, L)
                ra[i, sl] = ra[i, sl] + rb[i, sl]
            return carry

        lax.fori_loop(0, CHUNK, add_row, 0, unroll=4)
        wpending[b] = pltpu.async_copy(
            ra, out_hbm.at[pl.ds(base_of(k), CHUNK)], sem_w[b])
    for b in range(2):
        if wpending[b] is not None:
            wpending[b].wait()


@jax.jit
def _encode(in_deg, out_deg, in_table, out_table):
    mesh = plsc.VectorSubcoreMesh(core_axis_name="c", subcore_axis_name="s")
    kfn = pl.kernel(
        _body,
        out_type=jax.ShapeDtypeStruct((N_NODES, D), jnp.float32),
        mesh=mesh,
        scratch_types=[
            pltpu.VMEM((CHUNK,), jnp.int32),
            pltpu.VMEM((CHUNK,), jnp.int32),
            pltpu.VMEM((CHUNK,), jnp.int32),
            pltpu.VMEM((CHUNK,), jnp.int32),
            pltpu.VMEM((CHUNK, D), jnp.float32),
            pltpu.VMEM((CHUNK, D), jnp.float32),
            pltpu.VMEM((CHUNK, D), jnp.float32),
            pltpu.VMEM((CHUNK, D), jnp.float32),
            pltpu.SemaphoreType.DMA,
            pltpu.SemaphoreType.DMA,
            pltpu.SemaphoreType.DMA,
            pltpu.SemaphoreType.DMA,
        ],
    )
    return kfn(in_deg, out_deg, in_table, out_table)


def kernel(in_degrees, out_degrees, in_table, out_table):
    return _encode(in_degrees.astype(jnp.int32), out_degrees.astype(jnp.int32),
                   in_table, out_table)


# speedup vs baseline: 2.5140x; 1.1234x over previous
import jax
import jax.numpy as jnp
from jax import lax
from jax.experimental import pallas as pl
from jax.experimental.pallas import tpu as pltpu, tpu_sc as plsc

MAX_DEG = 512
D = 128
N_NODES = 100000

NC = 2
NS = 16
NW = NC * NS
L = 16

CHUNK = 128
K = 25
LAST_BASE = N_NODES - CHUNK


def _body(in_deg, out_deg, in_table, out_table, out_hbm, *scr):
    idx = [(scr[0], scr[1]), (scr[2], scr[3])]
    rows = [(scr[4], scr[5]), (scr[6], scr[7])]
    sem_g = [scr[8], scr[9]]
    sem_w = [scr[10], scr[11]]
    sem_i = [scr[12], scr[13]]

    wid = lax.axis_index("s") * NC + lax.axis_index("c")

    ipending = [None, None]
    pending = [None, None]
    wpending = [None, None]

    def base_of(k):
        return lax.min((k * NW + wid) * CHUNK, LAST_BASE)

    def fetch_idx(k):
        b = k % 2
        ia, ib = idx[b]
        base = base_of(k)
        ca = pltpu.async_copy(in_deg.at[pl.ds(base, CHUNK)], ia, sem_i[b])
        cb = pltpu.async_copy(out_deg.at[pl.ds(base, CHUNK)], ib, sem_i[b])
        ipending[b] = (ca, cb)

    def stage(k):
        b = k % 2
        ia, ib = idx[b]
        ra, rb = rows[b]
        if wpending[b] is not None:
            wpending[b].wait()
            wpending[b] = None
        ca, cb = ipending[b]
        ca.wait()
        cb.wait()
        ipending[b] = None
        for j in range(CHUNK // L):
            sl = pl.ds(j * L, L)
            ia[sl] = jnp.clip(ia[sl], 0, MAX_DEG)
            ib[sl] = jnp.clip(ib[sl], 0, MAX_DEG)
        cpa = pltpu.async_copy(in_table.at[ia], ra, sem_g[b])
        cpb = pltpu.async_copy(out_table.at[ib], rb, sem_g[b])
        pending[b] = (cpa, cpb)

    fetch_idx(0)
    fetch_idx(1)
    stage(0)
    for k in range(K):
        b = k % 2
        ra, rb = rows[b]
        if k + 1 < K:
            stage(k + 1)
        cpa, cpb = pending[b]
        cpa.wait()
        cpb.wait()
        if k + 2 < K:
            fetch_idx(k + 2)

        def add_row(i, carry):
            for j in range(D // L):
                sl = pl.ds(j * L, L)
                ra[i, sl] = ra[i, sl] + rb[i, sl]
            return carry

        lax.fori_loop(0, CHUNK, add_row, 0, unroll=4)
        wpending[b] = pltpu.async_copy(
            ra, out_hbm.at[pl.ds(base_of(k), CHUNK)], sem_w[b])
    for b in range(2):
        if wpending[b] is not None:
            wpending[b].wait()


@jax.jit
def _encode(in_deg, out_deg, in_table, out_table):
    mesh = plsc.VectorSubcoreMesh(core_axis_name="c", subcore_axis_name="s")
    kfn = pl.kernel(
        _body,
        out_type=jax.ShapeDtypeStruct((N_NODES, D), jnp.float32),
        mesh=mesh,
        scratch_types=[
            pltpu.VMEM((CHUNK,), jnp.int32),
            pltpu.VMEM((CHUNK,), jnp.int32),
            pltpu.VMEM((CHUNK,), jnp.int32),
            pltpu.VMEM((CHUNK,), jnp.int32),
            pltpu.VMEM((CHUNK, D), jnp.float32),
            pltpu.VMEM((CHUNK, D), jnp.float32),
            pltpu.VMEM((CHUNK, D), jnp.float32),
            pltpu.VMEM((CHUNK, D), jnp.float32),
            pltpu.SemaphoreType.DMA,
            pltpu.SemaphoreType.DMA,
            pltpu.SemaphoreType.DMA,
            pltpu.SemaphoreType.DMA,
            pltpu.SemaphoreType.DMA,
            pltpu.SemaphoreType.DMA,
        ],
    )
    return kfn(in_deg, out_deg, in_table, out_table)


def kernel(in_degrees, out_degrees, in_table, out_table):
    return _encode(in_degrees.astype(jnp.int32), out_degrees.astype(jnp.int32),
                   in_table, out_table)


# speedup vs baseline: 2.5519x; 1.0151x over previous
import jax
import jax.numpy as jnp
from jax import lax
from jax.experimental import pallas as pl
from jax.experimental.pallas import tpu as pltpu, tpu_sc as plsc

MAX_DEG = 512
D = 128
N_NODES = 100000

NC = 2
NS = 16
NW = NC * NS
L = 16

CHUNK = 128
K = 25
LAST_BASE = N_NODES - CHUNK


def _body(in_deg, out_deg, in_table, out_table, out_hbm, *scr):
    idx = [(scr[0], scr[1]), (scr[2], scr[3])]
    rows = [(scr[4], scr[5]), (scr[6], scr[7])]
    sem_g = [scr[8], scr[9]]
    sem_w = [scr[10], scr[11]]
    sem_i = [scr[12], scr[13]]
    tab_in, tab_out = scr[14], scr[15]

    sid = lax.axis_index("s")
    wid = sid * NC + lax.axis_index("c")

    @pl.when(sid == 0)
    def _load_tables():
        pltpu.sync_copy(in_table, tab_in)
        pltpu.sync_copy(out_table, tab_out)

    plsc.subcore_barrier()

    ipending = [None, None]
    pending = [None, None]
    wpending = [None, None]

    def base_of(k):
        return lax.min((k * NW + wid) * CHUNK, LAST_BASE)

    def fetch_idx(k):
        b = k % 2
        ia, ib = idx[b]
        base = base_of(k)
        ca = pltpu.async_copy(in_deg.at[pl.ds(base, CHUNK)], ia, sem_i[b])
        cb = pltpu.async_copy(out_deg.at[pl.ds(base, CHUNK)], ib, sem_i[b])
        ipending[b] = (ca, cb)

    def stage(k):
        b = k % 2
        ia, ib = idx[b]
        ra, rb = rows[b]
        if wpending[b] is not None:
            wpending[b].wait()
            wpending[b] = None
        ca, cb = ipending[b]
        ca.wait()
        cb.wait()
        ipending[b] = None
        for j in range(CHUNK // L):
            sl = pl.ds(j * L, L)
            ia[sl] = jnp.clip(ia[sl], 0, MAX_DEG)
            ib[sl] = jnp.clip(ib[sl], 0, MAX_DEG)
        cpa = pltpu.async_copy(tab_in.at[ia], ra, sem_g[b])
        cpb = pltpu.async_copy(tab_out.at[ib], rb, sem_g[b])
        pending[b] = (cpa, cpb)

    fetch_idx(0)
    fetch_idx(1)
    stage(0)
    for k in range(K):
        b = k % 2
        ra, rb = rows[b]
        if k + 1 < K:
            stage(k + 1)
        cpa, cpb = pending[b]
        cpa.wait()
        cpb.wait()
        if k + 2 < K:
            fetch_idx(k + 2)

        def add_row(i, carry):
            for j in range(D // L):
                sl = pl.ds(j * L, L)
                ra[i, sl] = ra[i, sl] + rb[i, sl]
            return carry

        lax.fori_loop(0, CHUNK, add_row, 0, unroll=4)
        wpending[b] = pltpu.async_copy(
            ra, out_hbm.at[pl.ds(base_of(k), CHUNK)], sem_w[b])
    for b in range(2):
        if wpending[b] is not None:
            wpending[b].wait()


@jax.jit
def _encode(in_deg, out_deg, in_table, out_table):
    mesh = plsc.VectorSubcoreMesh(core_axis_name="c", subcore_axis_name="s")
    kfn = pl.kernel(
        _body,
        out_type=jax.ShapeDtypeStruct((N_NODES, D), jnp.float32),
        mesh=mesh,
        scratch_types=[
            pltpu.VMEM((CHUNK,), jnp.int32),
            pltpu.VMEM((CHUNK,), jnp.int32),
            pltpu.VMEM((CHUNK,), jnp.int32),
            pltpu.VMEM((CHUNK,), jnp.int32),
            pltpu.VMEM((CHUNK, D), jnp.float32),
            pltpu.VMEM((CHUNK, D), jnp.float32),
            pltpu.VMEM((CHUNK, D), jnp.float32),
            pltpu.VMEM((CHUNK, D), jnp.float32),
            pltpu.SemaphoreType.DMA,
            pltpu.SemaphoreType.DMA,
            pltpu.SemaphoreType.DMA,
            pltpu.SemaphoreType.DMA,
            pltpu.SemaphoreType.DMA,
            pltpu.SemaphoreType.DMA,
            pltpu.VMEM_SHARED((MAX_DEG + 1, D), jnp.float32),
            pltpu.VMEM_SHARED((MAX_DEG + 1, D), jnp.float32),
        ],
    )
    return kfn(in_deg, out_deg, in_table, out_table)


def kernel(in_degrees, out_degrees, in_table, out_table):
    return _encode(in_degrees.astype(jnp.int32), out_degrees.astype(jnp.int32),
                   in_table, out_table)
